# baseline (device time: 90140 ns/iter reference)
import functools

import jax
import jax.numpy as jnp
from jax import lax
from jax.experimental import pallas as pl
from jax.experimental.pallas import tpu as pltpu

N_DEV = 8
N_SRC = 4
B, SQ, D = 2, 128, 512
HL, DH = 4, 64
SKV_SH = 128
QB = 64


def kernel(x, Wq, K_ext, V_ext, Wo):
    def body(
        x_ref, wq_ref, k_ref, v_ref, wo_ref, out_ref,
        kbuf, vbuf, pbuf, rbuf1, rbuf2, rbuf3,
        ksend, vsend, krecv, vrecv, rssend, rsrecv, agsend, agrecv,
    ):
        my = lax.axis_index("i")
        is_even = (my % 2) == 0
        my_m = my // 2

        bar = pltpu.get_barrier_semaphore()
        for d in range(1, N_DEV):
            pl.semaphore_signal(
                bar, inc=1, device_id=((my + d) % N_DEV,),
                device_id_type=pl.DeviceIdType.MESH,
            )
        pl.semaphore_wait(bar, N_DEV - 1)

        @pl.when(is_even)
        def _():
            for d in range(1, N_DEV):
                dst = (my + d) % N_DEV
                for src_r, buf, ssem, rsem in (
                    (k_ref, kbuf, ksend, krecv),
                    (v_ref, vbuf, vsend, vrecv),
                ):
                    rdma = pltpu.make_async_remote_copy(
                        src_ref=src_r.at[:, :, pl.ds(dst * HL, HL), :],
                        dst_ref=buf.at[my_m],
                        send_sem=ssem.at[d],
                        recv_sem=rsem.at[my_m],
                        device_id=(dst,),
                        device_id_type=pl.DeviceIdType.MESH,
                    )
                    rdma.start()
            kbuf[my_m] = k_ref[:, :, pl.ds(my * HL, HL), :]
            vbuf[my_m] = v_ref[:, :, pl.ds(my * HL, HL), :]

        q = [
            jnp.dot(x_ref[b], wq_ref[:, :], preferred_element_type=jnp.float32)
            for b in range(B)
        ]

        for m in range(N_SRC):
            @pl.when(jnp.logical_not(jnp.logical_and(is_even, my_m == m)))
            def _():
                for buf, rsem in ((kbuf, krecv), (vbuf, vrecv)):
                    rd = pltpu.make_async_remote_copy(
                        src_ref=k_ref.at[:, :, pl.ds(0, HL), :],
                        dst_ref=buf.at[m],
                        send_sem=ksend.at[0],
                        recv_sem=rsem.at[m],
                        device_id=(0,),
                        device_id_type=pl.DeviceIdType.MESH,
                    )
                    rd.wait_recv()

        for b in range(B):
            for blk in range(2):
                ctxs = []
                for h in range(HL):
                    qb = q[b][blk * QB : (blk + 1) * QB, h * DH : (h + 1) * DH]
                    kk = kbuf[:, b, blk * QB : (blk + 1) * QB, h, :].reshape(
                        N_SRC * QB, DH
                    )
                    vv = vbuf[:, b, blk * QB : (blk + 1) * QB, h, :].reshape(
                        N_SRC * QB, DH
                    )
                    s = (
                        lax.dot_general(
                            qb, kk, (((1,), (1,)), ((), ())),
                            preferred_element_type=jnp.float32,
                        )
                        * 0.125
                    )
                    s = s - jnp.max(s, axis=-1, keepdims=True)
                    w = jnp.exp(s)
                    w = w / jnp.sum(w, axis=-1, keepdims=True)
                    ctxs.append(
                        jnp.dot(w, vv, preferred_element_type=jnp.float32)
                    )
                pbuf[b, blk * QB : (blk + 1) * QB, :] = jnp.dot(
                    jnp.concatenate(ctxs, axis=1),
                    wo_ref[:, :],
                    preferred_element_type=jnp.float32,
                )

        b0 = my % 2
        b1 = (my // 2) % 2
        b2 = my // 4
        o1 = b2 * 64
        o2 = o1 + b1 * 32
        o3 = o2 + b0 * 16

        rs_rounds = [
            (4, (1 - b2) * 64, o1, 64, rbuf1),
            (3, o1 + (1 - b1) * 32, o2, 32, rbuf2),
            (1, o2 + (1 - b0) * 16, o3, 16, rbuf3),
        ]
        for r, (mask, soff, koff, w, rbuf) in enumerate(rs_rounds):
            partner = jnp.bitwise_xor(my, mask)
            rdma = pltpu.make_async_remote_copy(
                src_ref=pbuf.at[:, pl.ds(soff, w), :],
                dst_ref=rbuf,
                send_sem=rssend.at[r],
                recv_sem=rsrecv.at[r],
                device_id=(partner,),
                device_id_type=pl.DeviceIdType.MESH,
            )
            rdma.start()
            rdma.wait()
            pbuf[:, pl.ds(koff, w), :] = pbuf[:, pl.ds(koff, w), :] + rbuf[:, :, :]

        out_ref[:, pl.ds(o3, 16), :] = pbuf[:, pl.ds(o3, 16), :]

        ag_rounds = [(1, o3, 16), (3, o2, 32), (4, o1, 64)]
        for r, (mask, off, w) in enumerate(ag_rounds):
            partner = jnp.bitwise_xor(my, mask)
            rdma = pltpu.make_async_remote_copy(
                src_ref=out_ref.at[:, pl.ds(off, w), :],
                dst_ref=out_ref.at[:, pl.ds(off, w), :],
                send_sem=agsend.at[r],
                recv_sem=agrecv.at[r],
                device_id=(partner,),
                device_id_type=pl.DeviceIdType.MESH,
            )
            rdma.start()
            rdma.wait()

        @pl.when(is_even)
        def _():
            for d in range(1, N_DEV):
                for src_r, buf, ssem, rsem in (
                    (k_ref, kbuf, ksend, krecv),
                    (v_ref, vbuf, vsend, vrecv),
                ):
                    rd = pltpu.make_async_remote_copy(
                        src_ref=src_r.at[:, :, pl.ds(0, HL), :],
                        dst_ref=buf.at[0],
                        send_sem=ssem.at[d],
                        recv_sem=rsem.at[0],
                        device_id=(0,),
                        device_id_type=pl.DeviceIdType.MESH,
                    )
                    rd.wait_send()

        @functools.partial(pl.run_scoped, sem2=pltpu.SemaphoreType.REGULAR)
        def _(sem2):
            for d in range(1, N_DEV):
                pl.semaphore_signal(
                    sem2, inc=1, device_id=((my + d) % N_DEV,),
                    device_id_type=pl.DeviceIdType.MESH,
                )
            pl.semaphore_wait(sem2, N_DEV - 1)

    return pl.pallas_call(
        body,
        out_shape=jax.ShapeDtypeStruct((B, SQ, D), jnp.float32),
        in_specs=[pl.BlockSpec(memory_space=pltpu.VMEM)] * 5,
        out_specs=pl.BlockSpec(memory_space=pltpu.VMEM),
        scratch_shapes=[
            pltpu.VMEM((N_SRC, B, SKV_SH, HL, DH), jnp.float32),
            pltpu.VMEM((N_SRC, B, SKV_SH, HL, DH), jnp.float32),
            pltpu.VMEM((B, SQ, D), jnp.float32),
            pltpu.VMEM((B, 64, D), jnp.float32),
            pltpu.VMEM((B, 32, D), jnp.float32),
            pltpu.VMEM((B, 16, D), jnp.float32),
            pltpu.SemaphoreType.DMA((N_DEV,)),
            pltpu.SemaphoreType.DMA((N_DEV,)),
            pltpu.SemaphoreType.DMA((N_SRC,)),
            pltpu.SemaphoreType.DMA((N_SRC,)),
            pltpu.SemaphoreType.DMA((3,)),
            pltpu.SemaphoreType.DMA((3,)),
            pltpu.SemaphoreType.DMA((3,)),
            pltpu.SemaphoreType.DMA((3,)),
        ],
        compiler_params=pltpu.CompilerParams(collective_id=0),
    )(x, Wq, K_ext, V_ext, Wo)


# device time: 41229 ns/iter; 2.1863x vs baseline; 2.1863x over previous
import functools
import os

import jax
import jax.numpy as jnp
from jax import lax
from jax.experimental import pallas as pl
from jax.experimental.pallas import tpu as pltpu

_DO_KV = os.environ.get("ABL_KV", "1") == "1"
_DO_AR = os.environ.get("ABL_AR", "1") == "1"
_DO_ATTN = os.environ.get("ABL_ATTN", "1") == "1"

N_DEV = 8
N_SRC = 4
B, SQ, D = 2, 128, 512
HL, DH = 4, 64
HCH = HL * DH
SKV_SH = 128
QB = 64


def kernel(x, Wq, K_ext, V_ext, Wo):
    K2 = K_ext.reshape(B, SKV_SH, 32 * DH)
    V2 = V_ext.reshape(B, SKV_SH, 32 * DH)

    def body(
        x_ref, wq_ref, k_ref, v_ref, wo_ref, out_ref,
        kbuf, vbuf, pbuf, rsbuf, kstage, vstage,
        ksend, vsend, krecv, vrecv, rssend, rsrecv, agsend, agrecv,
        rlrecv, rfsend,
    ):
        my = lax.axis_index("i")
        is_even = (my % 2) == 0
        my_m = my // 2

        bar = pltpu.get_barrier_semaphore()
        for d in range(1, N_DEV):
            pl.semaphore_signal(
                bar, inc=1, device_id=((my + d) % N_DEV,),
                device_id_type=pl.DeviceIdType.MESH,
            )
        pl.semaphore_wait(bar, N_DEV - 1)

        if _DO_KV:
            @pl.when(is_even)
            def _():
                off = pl.multiple_of(my * HCH, HCH)
                for i, (src_r, buf, ssem, rsem, stage) in enumerate(
                    (
                        (k_ref, kbuf, ksend, krecv, kstage),
                        (v_ref, vbuf, vsend, vrecv, vstage),
                    )
                ):
                    pltpu.make_async_remote_copy(
                        src_ref=src_r.at[:, :, pl.ds(off, HCH)],
                        dst_ref=buf.at[my_m],
                        send_sem=ssem.at[0],
                        recv_sem=rsem.at[my_m],
                        device_id=(my,),
                        device_id_type=pl.DeviceIdType.MESH,
                    ).start()
                    for mask in range(1, N_DEV):
                        dst = jnp.bitwise_xor(my, mask)
                        doff = pl.multiple_of(dst * HCH, HCH)
                        if mask == 2:
                            relay = jnp.bitwise_xor(my, 3)
                            pltpu.make_async_remote_copy(
                                src_ref=src_r.at[:, :, pl.ds(doff, HCH)],
                                dst_ref=stage,
                                send_sem=ssem.at[mask],
                                recv_sem=rlrecv.at[i],
                                device_id=(relay,),
                                device_id_type=pl.DeviceIdType.MESH,
                            ).start()
                            continue
                        pltpu.make_async_remote_copy(
                            src_ref=src_r.at[:, :, pl.ds(doff, HCH)],
                            dst_ref=buf.at[my_m],
                            send_sem=ssem.at[mask],
                            recv_sem=rsem.at[my_m],
                            device_id=(dst,),
                            device_id_type=pl.DeviceIdType.MESH,
                        ).start()

        q = [
            jnp.dot(x_ref[b], wq_ref[:, :], preferred_element_type=jnp.float32)
            for b in range(B)
        ]

        if _DO_KV:
            @pl.when(jnp.logical_not(is_even))
            def _():
                src_p = jnp.bitwise_xor(my, 3)
                m_src = src_p // 2
                fwd_dst = jnp.bitwise_xor(my, 1)
                for i, (stage, buf, rsem_arr) in enumerate(
                    ((kstage, kbuf, krecv), (vstage, vbuf, vrecv))
                ):
                    rd = pltpu.make_async_remote_copy(
                        src_ref=stage,
                        dst_ref=stage,
                        send_sem=rfsend.at[i],
                        recv_sem=rlrecv.at[i],
                        device_id=(0,),
                        device_id_type=pl.DeviceIdType.MESH,
                    )
                    rd.wait_recv()
                    pltpu.make_async_remote_copy(
                        src_ref=stage,
                        dst_ref=buf.at[m_src],
                        send_sem=rfsend.at[i],
                        recv_sem=rsem_arr.at[m_src],
                        device_id=(fwd_dst,),
                        device_id_type=pl.DeviceIdType.MESH,
                    ).start()

        def wait_chunk(buf, rsem, m):
            rd = pltpu.make_async_remote_copy(
                src_ref=k_ref.at[:, :, pl.ds(0, HCH)],
                dst_ref=buf.at[m],
                send_sem=ksend.at[0],
                recv_sem=rsem.at[m],
                device_id=(0,),
                device_id_type=pl.DeviceIdType.MESH,
            )
            rd.wait_recv()

        if not _DO_ATTN:
            for m in range(N_SRC) if _DO_KV else ():
                wait_chunk(kbuf, krecv, m)
                wait_chunk(vbuf, vrecv, m)
            for b in range(B):
                pbuf[b] = jnp.dot(
                    q[b], wo_ref[:, :], preferred_element_type=jnp.float32
                )
        else:
            s = [
                [[[None] * N_SRC for _ in range(HL)] for _ in range(2)]
                for _ in range(B)
            ]
            for m in range(N_SRC):
                if _DO_KV:
                    wait_chunk(kbuf, krecv, m)
                for b in range(B):
                    for blk in range(2):
                        kblk = kbuf[m, b, blk * QB : (blk + 1) * QB, :]
                        for h in range(HL):
                            qb = q[b][
                                blk * QB : (blk + 1) * QB, h * DH : (h + 1) * DH
                            ]
                            kk = kblk[:, h * DH : (h + 1) * DH]
                            s[b][blk][h][m] = (
                                lax.dot_general(
                                    qb, kk, (((1,), (1,)), ((), ())),
                                    preferred_element_type=jnp.float32,
                                )
                                * 0.125
                            )
            wts = [[[None] * HL for _ in range(2)] for _ in range(B)]
            dens = [[[None] * HL for _ in range(2)] for _ in range(B)]
            for b in range(B):
                for blk in range(2):
                    for h in range(HL):
                        sl = s[b][blk][h]
                        gmax = functools.reduce(
                            jnp.maximum,
                            [jnp.max(t, axis=-1, keepdims=True) for t in sl],
                        )
                        wl = [jnp.exp(t - gmax) for t in sl]
                        dens[b][blk][h] = functools.reduce(
                            jnp.add,
                            [jnp.sum(t, axis=-1, keepdims=True) for t in wl],
                        )
                        wts[b][blk][h] = wl
            ctx = [[[None] * HL for _ in range(2)] for _ in range(B)]
            for m in range(N_SRC):
                if _DO_KV:
                    wait_chunk(vbuf, vrecv, m)
                for b in range(B):
                    for blk in range(2):
                        vblk = vbuf[m, b, blk * QB : (blk + 1) * QB, :]
                        for h in range(HL):
                            o = jnp.dot(
                                wts[b][blk][h][m],
                                vblk[:, h * DH : (h + 1) * DH],
                                preferred_element_type=jnp.float32,
                            )
                            ctx[b][blk][h] = (
                                o if ctx[b][blk][h] is None else ctx[b][blk][h] + o
                            )
            for blk in range(2):
                for b in range(B):
                    cc = jnp.concatenate(
                        [ctx[b][blk][h] / dens[b][blk][h] for h in range(HL)],
                        axis=1,
                    )
                    pbuf[b, blk * QB : (blk + 1) * QB, :] = jnp.dot(
                        cc, wo_ref[:, :], preferred_element_type=jnp.float32
                    )
                if _DO_AR:
                    for dst in range(blk * 4, blk * 4 + 4):
                        @pl.when(my != dst)
                        def _(dst=dst):
                            pltpu.make_async_remote_copy(
                                src_ref=pbuf.at[:, pl.ds(dst * 16, 16), :],
                                dst_ref=rsbuf.at[my],
                                send_sem=rssend.at[dst],
                                recv_sem=rsrecv.at[my],
                                device_id=(dst,),
                                device_id_type=pl.DeviceIdType.MESH,
                            ).start()

        if not _DO_AR:
            out_ref[...] = pbuf[...]
        if _DO_AR:
            myoff = pl.multiple_of(my * 16, 16)
            if not _DO_ATTN:
                for dst in range(N_DEV):
                    @pl.when(my != dst)
                    def _(dst=dst):
                        pltpu.make_async_remote_copy(
                            src_ref=pbuf.at[:, pl.ds(dst * 16, 16), :],
                            dst_ref=rsbuf.at[my],
                            send_sem=rssend.at[dst],
                            recv_sem=rsrecv.at[my],
                            device_id=(dst,),
                            device_id_type=pl.DeviceIdType.MESH,
                        ).start()
            for d in range(1, N_DEV):
                src = (my + d) % N_DEV
                rd = pltpu.make_async_remote_copy(
                    src_ref=pbuf.at[:, pl.ds(0, 16), :],
                    dst_ref=rsbuf.at[src],
                    send_sem=rssend.at[0],
                    recv_sem=rsrecv.at[src],
                    device_id=(0,),
                    device_id_type=pl.DeviceIdType.MESH,
                )
                rd.wait_recv()
            tot = pbuf[:, pl.ds(myoff, 16), :]
            for d in range(1, N_DEV):
                tot = tot + rsbuf[(my + d) % N_DEV]
            out_ref[:, pl.ds(myoff, 16), :] = tot

            for d in range(1, N_DEV):
                dst = (my + d) % N_DEV
                rdma = pltpu.make_async_remote_copy(
                    src_ref=out_ref.at[:, pl.ds(myoff, 16), :],
                    dst_ref=out_ref.at[:, pl.ds(myoff, 16), :],
                    send_sem=agsend.at[d],
                    recv_sem=agrecv.at[my],
                    device_id=(dst,),
                    device_id_type=pl.DeviceIdType.MESH,
                )
                rdma.start()
            for d in range(1, N_DEV):
                src = (my + d) % N_DEV
                rd = pltpu.make_async_remote_copy(
                    src_ref=pbuf.at[:, pl.ds(0, 16), :],
                    dst_ref=out_ref.at[:, pl.ds(pl.multiple_of(src * 16, 16), 16), :],
                    send_sem=agsend.at[0],
                    recv_sem=agrecv.at[src],
                    device_id=(0,),
                    device_id_type=pl.DeviceIdType.MESH,
                )
                rd.wait_recv()
            for dst in range(N_DEV):
                @pl.when(my != dst)
                def _(dst=dst):
                    pltpu.make_async_remote_copy(
                        src_ref=pbuf.at[:, pl.ds(0, 16), :],
                        dst_ref=rsbuf.at[0],
                        send_sem=rssend.at[dst],
                        recv_sem=rsrecv.at[0],
                        device_id=(0,),
                        device_id_type=pl.DeviceIdType.MESH,
                    ).wait_send()
            for d in range(1, N_DEV):
                rd = pltpu.make_async_remote_copy(
                    src_ref=pbuf.at[:, pl.ds(0, 16), :],
                    dst_ref=rsbuf.at[0],
                    send_sem=agsend.at[d],
                    recv_sem=rsrecv.at[0],
                    device_id=(0,),
                    device_id_type=pl.DeviceIdType.MESH,
                )
                rd.wait_send()

        if _DO_KV:
            @pl.when(is_even)
            def _():
                for d in range(N_DEV):
                    for src_r, buf, ssem, rsem in (
                        (k_ref, kbuf, ksend, krecv),
                        (v_ref, vbuf, vsend, vrecv),
                    ):
                        rd = pltpu.make_async_remote_copy(
                            src_ref=src_r.at[:, :, pl.ds(0, HCH)],
                            dst_ref=buf.at[0],
                            send_sem=ssem.at[d],
                            recv_sem=rsem.at[0],
                            device_id=(0,),
                            device_id_type=pl.DeviceIdType.MESH,
                        )
                        rd.wait_send()

            @pl.when(jnp.logical_not(is_even))
            def _():
                for i, stage in enumerate((kstage, vstage)):
                    rd = pltpu.make_async_remote_copy(
                        src_ref=stage,
                        dst_ref=kbuf.at[0],
                        send_sem=rfsend.at[i],
                        recv_sem=krecv.at[0],
                        device_id=(0,),
                        device_id_type=pl.DeviceIdType.MESH,
                    )
                    rd.wait_send()

        @functools.partial(pl.run_scoped, sem2=pltpu.SemaphoreType.REGULAR)
        def _(sem2):
            for d in range(1, N_DEV):
                pl.semaphore_signal(
                    sem2, inc=1, device_id=((my + d) % N_DEV,),
                    device_id_type=pl.DeviceIdType.MESH,
                )
            pl.semaphore_wait(sem2, N_DEV - 1)

    return pl.pallas_call(
        body,
        out_shape=jax.ShapeDtypeStruct((B, SQ, D), jnp.float32),
        in_specs=[pl.BlockSpec(memory_space=pltpu.VMEM)] * 5,
        out_specs=pl.BlockSpec(memory_space=pltpu.VMEM),
        scratch_shapes=[
            pltpu.VMEM((N_SRC, B, SKV_SH, HCH), jnp.float32),
            pltpu.VMEM((N_SRC, B, SKV_SH, HCH), jnp.float32),
            pltpu.VMEM((B, SQ, D), jnp.float32),
            pltpu.VMEM((N_DEV, B, 16, D), jnp.float32),
            pltpu.VMEM((B, SKV_SH, HCH), jnp.float32),
            pltpu.VMEM((B, SKV_SH, HCH), jnp.float32),
            pltpu.SemaphoreType.DMA((N_DEV,)),
            pltpu.SemaphoreType.DMA((N_DEV,)),
            pltpu.SemaphoreType.DMA((N_SRC,)),
            pltpu.SemaphoreType.DMA((N_SRC,)),
            pltpu.SemaphoreType.DMA((N_DEV,)),
            pltpu.SemaphoreType.DMA((N_DEV,)),
            pltpu.SemaphoreType.DMA((N_DEV,)),
            pltpu.SemaphoreType.DMA((N_DEV,)),
            pltpu.SemaphoreType.DMA((2,)),
            pltpu.SemaphoreType.DMA((2,)),
        ],
        compiler_params=pltpu.CompilerParams(collective_id=0),
    )(x, Wq, K2, V2, Wo)


# device time: 28868 ns/iter; 3.1225x vs baseline; 1.4282x over previous
import functools
import os

import jax
import jax.numpy as jnp
from jax import lax
from jax.experimental import pallas as pl
from jax.experimental.pallas import tpu as pltpu

_DO_KV = os.environ.get("ABL_KV", "1") == "1"
_DO_AR = os.environ.get("ABL_AR", "1") == "1"
_DO_ATTN = os.environ.get("ABL_ATTN", "1") == "1"

N_DEV = 8
N_SRC = 4
B, SQ, D = 2, 128, 512
HL, DH = 4, 64
HCH = HL * DH
SKV_SH = 128
QB = 64


def kernel(x, Wq, K_ext, V_ext, Wo):
    K2 = K_ext.astype(jnp.bfloat16).reshape(B, SKV_SH, 32 * DH)
    V2 = V_ext.astype(jnp.bfloat16).reshape(B, SKV_SH, 32 * DH)

    def body(
        x_ref, wq_ref, k_ref, v_ref, wo_ref, out_ref,
        kbuf, vbuf, pbuf, pbuf16, rsbuf, ag16, agbuf, kstage, vstage,
        ksend, vsend, krecv, vrecv, rssend, rsrecv, agsend, agrecv,
        rlrecv, rfsend,
    ):
        my = lax.axis_index("i")
        is_even = (my % 2) == 0
        my_m = my // 2

        bar = pltpu.get_barrier_semaphore()
        for d in range(1, N_DEV):
            pl.semaphore_signal(
                bar, inc=1, device_id=((my + d) % N_DEV,),
                device_id_type=pl.DeviceIdType.MESH,
            )
        pl.semaphore_wait(bar, N_DEV - 1)

        if _DO_KV:
            @pl.when(is_even)
            def _():
                off = pl.multiple_of(my * HCH, HCH)
                for i, (src_r, buf, ssem, rsem, stage) in enumerate(
                    (
                        (k_ref, kbuf, ksend, krecv, kstage),
                        (v_ref, vbuf, vsend, vrecv, vstage),
                    )
                ):
                    pltpu.make_async_remote_copy(
                        src_ref=src_r.at[:, :, pl.ds(off, HCH)],
                        dst_ref=buf.at[my_m],
                        send_sem=ssem.at[0],
                        recv_sem=rsem.at[my_m],
                        device_id=(my,),
                        device_id_type=pl.DeviceIdType.MESH,
                    ).start()
                    for mask in range(1, N_DEV):
                        dst = jnp.bitwise_xor(my, mask)
                        doff = pl.multiple_of(dst * HCH, HCH)
                        if mask == 2:
                            relay = jnp.bitwise_xor(my, 3)
                            pltpu.make_async_remote_copy(
                                src_ref=src_r.at[:, :, pl.ds(doff, HCH)],
                                dst_ref=stage,
                                send_sem=ssem.at[mask],
                                recv_sem=rlrecv.at[i],
                                device_id=(relay,),
                                device_id_type=pl.DeviceIdType.MESH,
                            ).start()
                            continue
                        pltpu.make_async_remote_copy(
                            src_ref=src_r.at[:, :, pl.ds(doff, HCH)],
                            dst_ref=buf.at[my_m],
                            send_sem=ssem.at[mask],
                            recv_sem=rsem.at[my_m],
                            device_id=(dst,),
                            device_id_type=pl.DeviceIdType.MESH,
                        ).start()

        q = [
            jnp.dot(x_ref[b], wq_ref[:, :], preferred_element_type=jnp.float32)
            for b in range(B)
        ]

        if _DO_KV:
            @pl.when(jnp.logical_not(is_even))
            def _():
                src_p = jnp.bitwise_xor(my, 3)
                m_src = src_p // 2
                fwd_dst = jnp.bitwise_xor(my, 1)
                for i, (stage, buf, rsem_arr) in enumerate(
                    ((kstage, kbuf, krecv), (vstage, vbuf, vrecv))
                ):
                    rd = pltpu.make_async_remote_copy(
                        src_ref=stage,
                        dst_ref=stage,
                        send_sem=rfsend.at[i],
                        recv_sem=rlrecv.at[i],
                        device_id=(0,),
                        device_id_type=pl.DeviceIdType.MESH,
                    )
                    rd.wait_recv()
                    pltpu.make_async_remote_copy(
                        src_ref=stage,
                        dst_ref=buf.at[m_src],
                        send_sem=rfsend.at[i],
                        recv_sem=rsem_arr.at[m_src],
                        device_id=(fwd_dst,),
                        device_id_type=pl.DeviceIdType.MESH,
                    ).start()

        def wait_chunk(buf, rsem, m):
            rd = pltpu.make_async_remote_copy(
                src_ref=k_ref.at[:, :, pl.ds(0, HCH)],
                dst_ref=buf.at[m],
                send_sem=ksend.at[0],
                recv_sem=rsem.at[m],
                device_id=(0,),
                device_id_type=pl.DeviceIdType.MESH,
            )
            rd.wait_recv()

        if not _DO_ATTN:
            for m in range(N_SRC) if _DO_KV else ():
                wait_chunk(kbuf, krecv, m)
                wait_chunk(vbuf, vrecv, m)
            for b in range(B):
                o = jnp.dot(q[b], wo_ref[:, :], preferred_element_type=jnp.float32)
                pbuf[b] = o
                pbuf16[b] = o.astype(jnp.bfloat16)
        else:
            s = [
                [[[None] * N_SRC for _ in range(HL)] for _ in range(2)]
                for _ in range(B)
            ]
            for m in range(N_SRC):
                if _DO_KV:
                    wait_chunk(kbuf, krecv, m)
                for b in range(B):
                    for blk in range(2):
                        kblk = kbuf[m, b, blk * QB : (blk + 1) * QB, :]
                        for h in range(HL):
                            qb = q[b][
                                blk * QB : (blk + 1) * QB, h * DH : (h + 1) * DH
                            ].astype(jnp.bfloat16)
                            kk = kblk[:, h * DH : (h + 1) * DH]
                            s[b][blk][h][m] = (
                                lax.dot_general(
                                    qb, kk, (((1,), (1,)), ((), ())),
                                    preferred_element_type=jnp.float32,
                                )
                                * 0.125
                            )
            wts = [[[None] * HL for _ in range(2)] for _ in range(B)]
            for b in range(B):
                for blk in range(2):
                    for h in range(HL):
                        sl = s[b][blk][h]
                        gmax = functools.reduce(
                            jnp.maximum,
                            [jnp.max(t, axis=-1, keepdims=True) for t in sl],
                        )
                        wl = [jnp.exp(t - gmax) for t in sl]
                        den = functools.reduce(
                            jnp.add,
                            [jnp.sum(t, axis=-1, keepdims=True) for t in wl],
                        )
                        wts[b][blk][h] = [
                            (t / den).astype(jnp.bfloat16) for t in wl
                        ]
            ctx = [[[None] * HL for _ in range(2)] for _ in range(B)]
            for m in range(N_SRC):
                if _DO_KV:
                    wait_chunk(vbuf, vrecv, m)
                for b in range(B):
                    for blk in range(2):
                        vblk = vbuf[m, b, blk * QB : (blk + 1) * QB, :]
                        for h in range(HL):
                            o = jnp.dot(
                                wts[b][blk][h][m],
                                vblk[:, h * DH : (h + 1) * DH],
                                preferred_element_type=jnp.float32,
                            )
                            ctx[b][blk][h] = (
                                o if ctx[b][blk][h] is None else ctx[b][blk][h] + o
                            )
            for blk in range(2):
                for b in range(B):
                    cc = jnp.concatenate(
                        [ctx[b][blk][h] for h in range(HL)], axis=1
                    )
                    o = jnp.dot(cc, wo_ref[:, :], preferred_element_type=jnp.float32)
                    pbuf[b, blk * QB : (blk + 1) * QB, :] = o
                    pbuf16[b, blk * QB : (blk + 1) * QB, :] = o.astype(jnp.bfloat16)
                if _DO_AR:
                    for dst in range(blk * 4, blk * 4 + 4):
                        @pl.when(my != dst)
                        def _(dst=dst):
                            pltpu.make_async_remote_copy(
                                src_ref=pbuf16.at[:, pl.ds(dst * 16, 16), :],
                                dst_ref=rsbuf.at[my],
                                send_sem=rssend.at[dst],
                                recv_sem=rsrecv.at[my],
                                device_id=(dst,),
                                device_id_type=pl.DeviceIdType.MESH,
                            ).start()

        if not _DO_AR:
            out_ref[...] = pbuf[...]
        if _DO_AR:
            myoff = pl.multiple_of(my * 16, 16)
            if not _DO_ATTN:
                for dst in range(N_DEV):
                    @pl.when(my != dst)
                    def _(dst=dst):
                        pltpu.make_async_remote_copy(
                            src_ref=pbuf16.at[:, pl.ds(dst * 16, 16), :],
                            dst_ref=rsbuf.at[my],
                            send_sem=rssend.at[dst],
                            recv_sem=rsrecv.at[my],
                            device_id=(dst,),
                            device_id_type=pl.DeviceIdType.MESH,
                        ).start()
            for d in range(1, N_DEV):
                src = (my + d) % N_DEV
                rd = pltpu.make_async_remote_copy(
                    src_ref=pbuf16.at[:, pl.ds(0, 16), :],
                    dst_ref=rsbuf.at[src],
                    send_sem=rssend.at[0],
                    recv_sem=rsrecv.at[src],
                    device_id=(0,),
                    device_id_type=pl.DeviceIdType.MESH,
                )
                rd.wait_recv()
            tot = pbuf[:, pl.ds(myoff, 16), :]
            for d in range(1, N_DEV):
                tot = tot + rsbuf[(my + d) % N_DEV]
            out_ref[:, pl.ds(myoff, 16), :] = tot
            ag16[...] = tot.astype(jnp.bfloat16)

            for d in range(1, N_DEV):
                dst = (my + d) % N_DEV
                rdma = pltpu.make_async_remote_copy(
                    src_ref=ag16,
                    dst_ref=agbuf.at[my],
                    send_sem=agsend.at[d],
                    recv_sem=agrecv.at[my],
                    device_id=(dst,),
                    device_id_type=pl.DeviceIdType.MESH,
                )
                rdma.start()
            for d in range(1, N_DEV):
                src = (my + d) % N_DEV
                rd = pltpu.make_async_remote_copy(
                    src_ref=ag16,
                    dst_ref=agbuf.at[src],
                    send_sem=agsend.at[0],
                    recv_sem=agrecv.at[src],
                    device_id=(0,),
                    device_id_type=pl.DeviceIdType.MESH,
                )
                rd.wait_recv()
                out_ref[:, pl.ds(pl.multiple_of(src * 16, 16), 16), :] = agbuf[
                    src
                ].astype(jnp.float32)
            for dst in range(N_DEV):
                @pl.when(my != dst)
                def _(dst=dst):
                    pltpu.make_async_remote_copy(
                        src_ref=pbuf16.at[:, pl.ds(0, 16), :],
                        dst_ref=rsbuf.at[0],
                        send_sem=rssend.at[dst],
                        recv_sem=rsrecv.at[0],
                        device_id=(0,),
                        device_id_type=pl.DeviceIdType.MESH,
                    ).wait_send()
            for d in range(1, N_DEV):
                rd = pltpu.make_async_remote_copy(
                    src_ref=ag16,
                    dst_ref=agbuf.at[0],
                    send_sem=agsend.at[d],
                    recv_sem=agrecv.at[0],
                    device_id=(0,),
                    device_id_type=pl.DeviceIdType.MESH,
                )
                rd.wait_send()

        if _DO_KV:
            @pl.when(is_even)
            def _():
                for d in range(N_DEV):
                    for src_r, buf, ssem, rsem in (
                        (k_ref, kbuf, ksend, krecv),
                        (v_ref, vbuf, vsend, vrecv),
                    ):
                        rd = pltpu.make_async_remote_copy(
                            src_ref=src_r.at[:, :, pl.ds(0, HCH)],
                            dst_ref=buf.at[0],
                            send_sem=ssem.at[d],
                            recv_sem=rsem.at[0],
                            device_id=(0,),
                            device_id_type=pl.DeviceIdType.MESH,
                        )
                        rd.wait_send()

            @pl.when(jnp.logical_not(is_even))
            def _():
                for i, stage in enumerate((kstage, vstage)):
                    rd = pltpu.make_async_remote_copy(
                        src_ref=stage,
                        dst_ref=kbuf.at[0],
                        send_sem=rfsend.at[i],
                        recv_sem=krecv.at[0],
                        device_id=(0,),
                        device_id_type=pl.DeviceIdType.MESH,
                    )
                    rd.wait_send()

        @functools.partial(pl.run_scoped, sem2=pltpu.SemaphoreType.REGULAR)
        def _(sem2):
            for d in range(1, N_DEV):
                pl.semaphore_signal(
                    sem2, inc=1, device_id=((my + d) % N_DEV,),
                    device_id_type=pl.DeviceIdType.MESH,
                )
            pl.semaphore_wait(sem2, N_DEV - 1)

    return pl.pallas_call(
        body,
        out_shape=jax.ShapeDtypeStruct((B, SQ, D), jnp.float32),
        in_specs=[pl.BlockSpec(memory_space=pltpu.VMEM)] * 5,
        out_specs=pl.BlockSpec(memory_space=pltpu.VMEM),
        scratch_shapes=[
            pltpu.VMEM((N_SRC, B, SKV_SH, HCH), jnp.bfloat16),
            pltpu.VMEM((N_SRC, B, SKV_SH, HCH), jnp.bfloat16),
            pltpu.VMEM((B, SQ, D), jnp.float32),
            pltpu.VMEM((B, SQ, D), jnp.bfloat16),
            pltpu.VMEM((N_DEV, B, 16, D), jnp.bfloat16),
            pltpu.VMEM((B, 16, D), jnp.bfloat16),
            pltpu.VMEM((N_DEV, B, 16, D), jnp.bfloat16),
            pltpu.VMEM((B, SKV_SH, HCH), jnp.bfloat16),
            pltpu.VMEM((B, SKV_SH, HCH), jnp.bfloat16),
            pltpu.SemaphoreType.DMA((N_DEV,)),
            pltpu.SemaphoreType.DMA((N_DEV,)),
            pltpu.SemaphoreType.DMA((N_SRC,)),
            pltpu.SemaphoreType.DMA((N_SRC,)),
            pltpu.SemaphoreType.DMA((N_DEV,)),
            pltpu.SemaphoreType.DMA((N_DEV,)),
            pltpu.SemaphoreType.DMA((N_DEV,)),
            pltpu.SemaphoreType.DMA((N_DEV,)),
            pltpu.SemaphoreType.DMA((2,)),
            pltpu.SemaphoreType.DMA((2,)),
        ],
        compiler_params=pltpu.CompilerParams(collective_id=0),
    )(x, Wq, K2, V2, Wo)
